# baseline (speedup 1.0000x reference)
import functools

import jax
import jax.numpy as jnp
from jax import lax
from jax.experimental import pallas as pl
from jax.experimental.pallas import tpu as pltpu
from jax.experimental.pallas import tpu_sc as plsc

VPAD = 8
NW = 32
TCH = 32


def kernel(seqs, species, vocab_emb, cat_emb, pos_emb):
    B, S = seqs.shape
    V, D = vocab_emb.shape
    rows_w = S // NW
    nch = rows_w // TCH

    seqs_flat = seqs.astype(jnp.int32).reshape(B * S)
    species32 = species.astype(jnp.int32)
    vocab_pad = jnp.concatenate(
        [vocab_emb, jnp.zeros((VPAD - V, D), jnp.float32)], axis=0
    )

    mesh = plsc.VectorSubcoreMesh(core_axis_name="c", subcore_axis_name="s")

    @functools.partial(
        pl.kernel,
        out_type=(
            jax.ShapeDtypeStruct((B * S, D), jnp.float32),
            jax.ShapeDtypeStruct((B, D), jnp.float32),
            jax.ShapeDtypeStruct((2 * B * VPAD, D), jnp.float32),
        ),
        mesh=mesh,
        scratch_types=[
            pltpu.VMEM((VPAD, D), jnp.float32),
            pltpu.VMEM((B, D), jnp.float32),
            pltpu.VMEM((B * VPAD, D), jnp.float32),
            pltpu.VMEM((TCH, D), jnp.float32),
            pltpu.VMEM((TCH, D), jnp.float32),
            pltpu.VMEM((B,), jnp.int32),
            pltpu.VMEM((TCH,), jnp.int32),
            pltpu.VMEM((TCH,), jnp.int32),
            pltpu.SemaphoreType.DMA,
        ],
    )
    def run(seqs_hbm, species_hbm, vocab_hbm, cat_hbm, pos_hbm,
            out_hbm, spe_hbm, comb_hbm,
            vocab_v, cat_v, comb_v, bufa, bufb, spidx_v, idx_v, gidx_v, sem):
        cid = lax.axis_index("c")
        sid = lax.axis_index("s")
        wid = cid * 16 + sid

        pltpu.sync_copy(species_hbm, spidx_v)
        pltpu.async_copy(cat_hbm.at[spidx_v], cat_v, sem).wait()
        pltpu.sync_copy(vocab_hbm, vocab_v)

        @pl.when(jnp.logical_and(cid == 0, sid == 0))
        def _():
            pltpu.sync_copy(cat_v, spe_hbm)

        @pl.when(sid == 0)
        def _():
            for b in range(B):
                for v in range(VPAD):

                    def _comb(j, carry, b=b, v=v):
                        sl = pl.ds(j * 16, 16)
                        comb_v[b * VPAD + v, sl] = vocab_v[v, sl] + cat_v[b, sl]
                        return carry

                    lax.fori_loop(0, D // 16, _comb, 0)

            pltpu.sync_copy(comb_v, comb_hbm.at[pl.ds(cid * (B * VPAD), B * VPAD)])

        plsc.subcore_barrier()

        base = wid * rows_w
        for t in range(nch):
            row0 = base + t * TCH
            pltpu.sync_copy(pos_hbm.at[pl.ds(row0, TCH)], bufb)
            for b in range(B):
                pltpu.sync_copy(seqs_hbm.at[pl.ds(b * S + row0, TCH)], idx_v)
                for h in range(TCH // 16):
                    sl = pl.ds(h * 16, 16)
                    gidx_v[sl] = idx_v[sl] + (b * VPAD + cid * (B * VPAD))
                pltpu.async_copy(comb_hbm.at[gidx_v], bufa, sem).wait()

                for r in range(TCH):

                    def _add(j, carry, r=r):
                        sl = pl.ds(j * 16, 16)
                        bufa[r, sl] = bufa[r, sl] + bufb[r, sl]
                        return carry

                    lax.fori_loop(0, D // 16, _add, 0)

                pltpu.sync_copy(bufa, out_hbm.at[pl.ds(b * S + row0, TCH)])

    total_flat, species_emb, _comb_scratch = run(
        seqs_flat, species32, vocab_pad, cat_emb, pos_emb
    )
    return (total_flat.reshape(B, S, D), species_emb)


# speedup vs baseline: 4.7861x; 4.7861x over previous
import functools

import jax
import jax.numpy as jnp
from jax.experimental import pallas as pl
from jax.experimental.pallas import tpu as pltpu
from jax.experimental.pallas import tpu_sc as plsc

VPAD = 8


def _species_sc(species32, cat_emb):
    B = species32.shape[0]
    D = cat_emb.shape[1]
    mesh = plsc.VectorSubcoreMesh(
        core_axis_name="c", subcore_axis_name="s", num_cores=1
    )

    @functools.partial(
        pl.kernel,
        out_type=jax.ShapeDtypeStruct((B, D), jnp.float32),
        mesh=mesh,
        scratch_types=[
            pltpu.VMEM((B,), jnp.int32),
            pltpu.VMEM((B, D), jnp.float32),
            pltpu.SemaphoreType.DMA,
        ],
    )
    def run(species_hbm, cat_hbm, out_hbm, idx_v, rows_v, sem):
        first = jnp.logical_and(
            jax.lax.axis_index("c") == 0, jax.lax.axis_index("s") == 0
        )

        @pl.when(first)
        def _():
            pltpu.sync_copy(species_hbm, idx_v)
            pltpu.async_copy(cat_hbm.at[idx_v], rows_v, sem).wait()
            pltpu.sync_copy(rows_v, out_hbm)

    return run(species32, cat_emb)


def _body(spe_idx_ref, seqs_ref, vocab_ref, cat_ref, pos_ref, out_ref, spe_out_ref):
    idx = seqs_ref[0, 0, :]
    n = idx.shape[0]
    iota = jax.lax.broadcasted_iota(jnp.int32, (n, VPAD), 1)
    oh = (idx[:, None] == iota).astype(jnp.float32)
    seq_emb = jnp.dot(oh, vocab_ref[...], preferred_element_type=jnp.float32)
    out_ref[...] = (seq_emb + cat_ref[0] + pos_ref[...])[None]
    spe_out_ref[...] = cat_ref[...]


def kernel(seqs, species, vocab_emb, cat_emb, pos_emb):
    B, S = seqs.shape
    V, D = vocab_emb.shape

    seqs3 = seqs.astype(jnp.int32).reshape(B, 1, S)
    species32 = species.astype(jnp.int32)
    vocab_pad = jnp.concatenate(
        [vocab_emb, jnp.zeros((VPAD - V, D), vocab_emb.dtype)], axis=0
    )
    cat3 = cat_emb.reshape(cat_emb.shape[0], 1, D)

    species_emb = _species_sc(species32, cat_emb)

    total, species_emb3 = pl.pallas_call(
        _body,
        grid_spec=pltpu.PrefetchScalarGridSpec(
            num_scalar_prefetch=1,
            grid=(B,),
            in_specs=[
                pl.BlockSpec((1, 1, S), lambda b, spe: (b, 0, 0)),
                pl.BlockSpec((VPAD, D), lambda b, spe: (0, 0)),
                pl.BlockSpec((1, 1, D), lambda b, spe: (spe[b], 0, 0)),
                pl.BlockSpec((S, D), lambda b, spe: (0, 0)),
            ],
            out_specs=[
                pl.BlockSpec((1, S, D), lambda b, spe: (b, 0, 0)),
                pl.BlockSpec((1, 1, D), lambda b, spe: (b, 0, 0)),
            ],
        ),
        out_shape=[
            jax.ShapeDtypeStruct((B, S, D), jnp.float32),
            jax.ShapeDtypeStruct((B, 1, D), jnp.float32),
        ],
        compiler_params=pltpu.CompilerParams(dimension_semantics=("arbitrary",)),
    )(species32, seqs3, vocab_pad, cat3, pos_emb)

    del species_emb3
    return (total, species_emb)
